# baseline (speedup 1.0000x reference)
import jax
import jax.numpy as jnp
from jax import lax
from jax.experimental import pallas as pl
from jax.experimental.pallas import tpu as pltpu, tpu_sc as plsc

_B = 16384
_D = 32
_CHUNK = 128
_NROW = 1000001


def _build_sc_call():
    info = plsc.get_sparse_core_info()
    nc, ns = info.num_cores, info.num_subcores
    nw = nc * ns
    bpw = _B // nw
    nchunk = bpw // _CHUNK

    mesh = plsc.VectorSubcoreMesh(core_axis_name="c", subcore_axis_name="s")

    def body(user_hbm, pos_hbm, neg_hbm, utab_hbm, itab_hbm,
             pos_out, neg_out,
             uidx_v, pidx_v, nidx_v, ubuf_v, pbuf_v, nbuf_v,
             posbuf_v, negbuf_v, sem):
        wid = lax.axis_index("s") * nc + lax.axis_index("c")
        ibase = wid * nchunk

        pltpu.sync_copy(user_hbm.at[pl.ds(ibase, nchunk)], uidx_v)
        pltpu.sync_copy(pos_hbm.at[pl.ds(ibase, nchunk)], pidx_v)
        pltpu.sync_copy(neg_hbm.at[pl.ds(ibase, nchunk)], nidx_v)

        def chunk(j, carry):
            copies = []
            for d in range(_D):
                copies.append(pltpu.async_copy(
                    utab_hbm.at[d].at[uidx_v.at[j]], ubuf_v.at[d], sem))
                copies.append(pltpu.async_copy(
                    itab_hbm.at[d].at[pidx_v.at[j]], pbuf_v.at[d], sem))
                copies.append(pltpu.async_copy(
                    itab_hbm.at[d].at[nidx_v.at[j]], nbuf_v.at[d], sem))
            for c in copies:
                c.wait()

            for k in range(_CHUNK // 16):
                sl = pl.ds(k * 16, 16)
                accp = jnp.zeros((16,), jnp.float32)
                accn = jnp.zeros((16,), jnp.float32)
                for d in range(_D):
                    u = ubuf_v[d, sl]
                    accp = accp + u * pbuf_v[d, sl]
                    accn = accn + u * nbuf_v[d, sl]
                posbuf_v[pl.ds(j * _CHUNK + k * 16, 16)] = accp
                negbuf_v[pl.ds(j * _CHUNK + k * 16, 16)] = accn
            return carry

        lax.fori_loop(0, nchunk, chunk, 0)

        obase = wid * bpw
        pltpu.sync_copy(posbuf_v, pos_out.at[pl.ds(obase, bpw)])
        pltpu.sync_copy(negbuf_v, neg_out.at[pl.ds(obase, bpw)])

    call = pl.kernel(
        body,
        out_type=(jax.ShapeDtypeStruct((_B,), jnp.float32),
                  jax.ShapeDtypeStruct((_B,), jnp.float32)),
        mesh=mesh,
        scratch_types=[
            pltpu.VMEM((_B // _CHUNK // 32, _CHUNK), jnp.int32),
            pltpu.VMEM((_B // _CHUNK // 32, _CHUNK), jnp.int32),
            pltpu.VMEM((_B // _CHUNK // 32, _CHUNK), jnp.int32),
            pltpu.VMEM((_D, _CHUNK), jnp.float32),
            pltpu.VMEM((_D, _CHUNK), jnp.float32),
            pltpu.VMEM((_D, _CHUNK), jnp.float32),
            pltpu.VMEM((bpw,), jnp.float32),
            pltpu.VMEM((bpw,), jnp.float32),
            pltpu.SemaphoreType.DMA,
        ],
        compiler_params=pltpu.CompilerParams(
            needs_layout_passes=False, use_tc_tiling_on_sc=False),
    )
    return call


def kernel(user, pos_item, neg_item, user_table, item_table):
    call = _build_sc_call()
    u2 = user.astype(jnp.int32).reshape(_B // _CHUNK, _CHUNK)
    p2 = pos_item.astype(jnp.int32).reshape(_B // _CHUNK, _CHUNK)
    n2 = neg_item.astype(jnp.int32).reshape(_B // _CHUNK, _CHUNK)
    ut = jnp.swapaxes(user_table, 0, 1)
    it = jnp.swapaxes(item_table, 0, 1)
    return call(u2, p2, n2, ut, it)


# speedup vs baseline: 15.3219x; 15.3219x over previous
import jax
import jax.numpy as jnp
from jax import lax
from jax.experimental import pallas as pl
from jax.experimental.pallas import tpu as pltpu, tpu_sc as plsc

_B = 16384
_D = 32
_CHUNK = 128
_NROW = 1000001
_NCOL = 7813
_LANES = _NCOL * 128
_SUPER = 8


def _build_detile_call():
    info = plsc.get_sparse_core_info()
    nc, ns = info.num_cores, info.num_subcores
    nw = nc * ns
    mesh = plsc.VectorSubcoreMesh(core_axis_name="c", subcore_axis_name="s")
    sw = _SUPER * 128

    def body(utab_hbm, itab_hbm, ulin_out, ilin_out, blk_a, blk_b, tblk,
             rsem, wsem):
        wid = lax.axis_index("s") * nc + lax.axis_index("c")
        lo = wid * _NCOL // nw
        hi = (wid + 1) * _NCOL // nw
        nsup = (hi - lo) // _SUPER

        dummy_big = utab_hbm.at[:, pl.ds(0, sw)]
        dummy_small = utab_hbm.at[:, pl.ds(0, 128)]

        for tab, out in ((utab_hbm, ulin_out), (itab_hbm, ilin_out)):
            def super_step(i, carry):
                base = (lo + i * _SUPER) * 128
                blk = blk_a
                pltpu.async_copy(
                    tab.at[:, pl.ds(base, sw)], blk, rsem).wait()
                for d in range(_D):
                    pltpu.async_copy(
                        blk.at[d], out.at[pl.ds(d * _LANES + base, sw)],
                        wsem)
                pltpu.make_async_copy(dummy_big, blk, wsem).wait()
                return carry

            lax.fori_loop(0, nsup, super_step, 0)

            def tail_step(i, carry):
                c = lo + nsup * _SUPER + i
                base = c * 128
                pltpu.async_copy(
                    tab.at[:, pl.ds(base, 128)], tblk, rsem).wait()
                for d in range(_D):
                    pltpu.async_copy(
                        tblk.at[d], out.at[pl.ds(d * _LANES + base, 128)],
                        wsem)
                pltpu.make_async_copy(dummy_small, tblk, wsem).wait()
                return carry

            lax.fori_loop(0, (hi - lo) - nsup * _SUPER, tail_step, 0)

    call = pl.kernel(
        body,
        out_type=(jax.ShapeDtypeStruct((_D * _LANES,), jnp.float32),
                  jax.ShapeDtypeStruct((_D * _LANES,), jnp.float32)),
        mesh=mesh,
        scratch_types=[
            pltpu.VMEM((_D, sw), jnp.float32),
            pltpu.VMEM((_D, sw), jnp.float32),
            pltpu.VMEM((_D, 128), jnp.float32),
            pltpu.SemaphoreType.DMA,
            pltpu.SemaphoreType.DMA,
        ],
        compiler_params=pltpu.CompilerParams(
            needs_layout_passes=False, use_tc_tiling_on_sc=True,
            disable_bounds_checks=True),
    )
    return call


def _build_score_call():
    info = plsc.get_sparse_core_info()
    nc, ns = info.num_cores, info.num_subcores
    nw = nc * ns
    bpw = _B // nw
    nchunk = bpw // _CHUNK

    mesh = plsc.VectorSubcoreMesh(core_axis_name="c", subcore_axis_name="s")

    def body(user_hbm, pos_hbm, neg_hbm, utab_hbm, itab_hbm,
             pos_out, neg_out,
             uidx_v, pidx_v, nidx_v, ubidx_v, pbidx_v, nbidx_v,
             ubuf_v, pbuf_v, nbuf_v,
             posbuf_v, negbuf_v, sem):
        wid = lax.axis_index("s") * nc + lax.axis_index("c")
        ibase = wid * nchunk

        pltpu.sync_copy(user_hbm.at[pl.ds(ibase, nchunk)], uidx_v)
        pltpu.sync_copy(pos_hbm.at[pl.ds(ibase, nchunk)], pidx_v)
        pltpu.sync_copy(neg_hbm.at[pl.ds(ibase, nchunk)], nidx_v)

        def chunk(j, carry):
            for d in range(_D):
                for k in range(_CHUNK // 16):
                    sl = pl.ds(k * 16, 16)
                    ubidx_v[d, sl] = uidx_v[j, sl] + d * _LANES
                    pbidx_v[d, sl] = pidx_v[j, sl] + d * _LANES
                    nbidx_v[d, sl] = nidx_v[j, sl] + d * _LANES
            copies = []
            for d in range(_D):
                copies.append(pltpu.async_copy(
                    utab_hbm.at[ubidx_v.at[d]], ubuf_v.at[d], sem))
                copies.append(pltpu.async_copy(
                    itab_hbm.at[pbidx_v.at[d]], pbuf_v.at[d], sem))
                copies.append(pltpu.async_copy(
                    itab_hbm.at[nbidx_v.at[d]], nbuf_v.at[d], sem))
            for c in copies:
                c.wait()

            for k in range(_CHUNK // 16):
                sl = pl.ds(k * 16, 16)
                accp = jnp.zeros((16,), jnp.float32)
                accn = jnp.zeros((16,), jnp.float32)
                for d in range(_D):
                    u = ubuf_v[d, sl]
                    accp = accp + u * pbuf_v[d, sl]
                    accn = accn + u * nbuf_v[d, sl]
                posbuf_v[pl.ds(j * _CHUNK + k * 16, 16)] = accp
                negbuf_v[pl.ds(j * _CHUNK + k * 16, 16)] = accn
            return carry

        lax.fori_loop(0, nchunk, chunk, 0)

        obase = wid * bpw
        pltpu.sync_copy(posbuf_v, pos_out.at[pl.ds(obase, bpw)])
        pltpu.sync_copy(negbuf_v, neg_out.at[pl.ds(obase, bpw)])

    call = pl.kernel(
        body,
        out_type=(jax.ShapeDtypeStruct((_B,), jnp.float32),
                  jax.ShapeDtypeStruct((_B,), jnp.float32)),
        mesh=mesh,
        scratch_types=[
            pltpu.VMEM((_B // _CHUNK // 32, _CHUNK), jnp.int32),
            pltpu.VMEM((_B // _CHUNK // 32, _CHUNK), jnp.int32),
            pltpu.VMEM((_B // _CHUNK // 32, _CHUNK), jnp.int32),
            pltpu.VMEM((_D, _CHUNK), jnp.int32),
            pltpu.VMEM((_D, _CHUNK), jnp.int32),
            pltpu.VMEM((_D, _CHUNK), jnp.int32),
            pltpu.VMEM((_D, _CHUNK), jnp.float32),
            pltpu.VMEM((_D, _CHUNK), jnp.float32),
            pltpu.VMEM((_D, _CHUNK), jnp.float32),
            pltpu.VMEM((_B // 32,), jnp.float32),
            pltpu.VMEM((_B // 32,), jnp.float32),
            pltpu.SemaphoreType.DMA,
        ],
        compiler_params=pltpu.CompilerParams(
            needs_layout_passes=False, use_tc_tiling_on_sc=False),
    )
    return call


def kernel(user, pos_item, neg_item, user_table, item_table):
    detile = _build_detile_call()
    score = _build_score_call()
    ut = jnp.swapaxes(user_table, 0, 1)
    it = jnp.swapaxes(item_table, 0, 1)
    ulin, ilin = detile(ut, it)
    u2 = user.astype(jnp.int32).reshape(_B // _CHUNK, _CHUNK)
    p2 = pos_item.astype(jnp.int32).reshape(_B // _CHUNK, _CHUNK)
    n2 = neg_item.astype(jnp.int32).reshape(_B // _CHUNK, _CHUNK)
    return score(u2, p2, n2, ulin, ilin)


# speedup vs baseline: 15.4892x; 1.0109x over previous
import jax
import jax.numpy as jnp
from jax import lax
from jax.experimental import pallas as pl
from jax.experimental.pallas import tpu as pltpu, tpu_sc as plsc

_B = 16384
_D = 32
_CHUNK = 128
_NROW = 1000001
_NCOL = 7813
_LANES = _NCOL * 128
_SUPER = 8


def _build_detile_call():
    info = plsc.get_sparse_core_info()
    nc, ns = info.num_cores, info.num_subcores
    nw = nc * ns
    mesh = plsc.VectorSubcoreMesh(core_axis_name="c", subcore_axis_name="s")
    sw = _SUPER * 128

    def body(utab_hbm, itab_hbm, ulin_out, ilin_out, blk_a, blk_b, tblk,
             rsem, wsem_a, wsem_b):
        wid = lax.axis_index("s") * nc + lax.axis_index("c")
        lo = wid * _NCOL // nw
        hi = (wid + 1) * _NCOL // nw
        nsup = (hi - lo) // _SUPER

        dummy_big = utab_hbm.at[:, pl.ds(0, sw)]
        dummy_small = utab_hbm.at[:, pl.ds(0, 128)]
        max_base = (_NCOL - _SUPER) * 128
        npair = nsup // 2

        for tab, out in ((utab_hbm, ulin_out), (itab_hbm, ilin_out)):
            pltpu.async_copy(tab.at[:, pl.ds(lo * 128, sw)], blk_a, rsem)

            def pair_step(i, carry):
                base0 = (lo + 2 * i * _SUPER) * 128
                base1 = base0 + sw
                base2 = jnp.minimum(base0 + 2 * sw, max_base)
                pltpu.make_async_copy(dummy_big, blk_a, rsem).wait()
                for d in range(_D):
                    pltpu.async_copy(
                        blk_a.at[d], out.at[pl.ds(d * _LANES + base0, sw)],
                        wsem_a)
                @pl.when(i > 0)
                def _():
                    pltpu.make_async_copy(dummy_big, blk_b, wsem_b).wait()
                pltpu.async_copy(tab.at[:, pl.ds(base1, sw)], blk_b, rsem)
                pltpu.make_async_copy(dummy_big, blk_b, rsem).wait()
                for d in range(_D):
                    pltpu.async_copy(
                        blk_b.at[d], out.at[pl.ds(d * _LANES + base1, sw)],
                        wsem_b)
                pltpu.make_async_copy(dummy_big, blk_a, wsem_a).wait()
                pltpu.async_copy(tab.at[:, pl.ds(base2, sw)], blk_a, rsem)
                return carry

            lax.fori_loop(0, npair, pair_step, 0)
            pltpu.make_async_copy(dummy_big, blk_a, rsem).wait()
            pltpu.make_async_copy(dummy_big, blk_b, wsem_b).wait()

            def tail_step(i, carry):
                c = lo + npair * 2 * _SUPER + i
                base = c * 128
                pltpu.async_copy(
                    tab.at[:, pl.ds(base, 128)], tblk, rsem).wait()
                for d in range(_D):
                    pltpu.async_copy(
                        tblk.at[d], out.at[pl.ds(d * _LANES + base, 128)],
                        wsem_a)
                pltpu.make_async_copy(dummy_small, tblk, wsem_a).wait()
                return carry

            lax.fori_loop(0, (hi - lo) - npair * 2 * _SUPER, tail_step, 0)

    call = pl.kernel(
        body,
        out_type=(jax.ShapeDtypeStruct((_D * _LANES,), jnp.float32),
                  jax.ShapeDtypeStruct((_D * _LANES,), jnp.float32)),
        mesh=mesh,
        scratch_types=[
            pltpu.VMEM((_D, sw), jnp.float32),
            pltpu.VMEM((_D, sw), jnp.float32),
            pltpu.VMEM((_D, 128), jnp.float32),
            pltpu.SemaphoreType.DMA,
            pltpu.SemaphoreType.DMA,
            pltpu.SemaphoreType.DMA,
        ],
        compiler_params=pltpu.CompilerParams(
            needs_layout_passes=False, use_tc_tiling_on_sc=True,
            disable_bounds_checks=True),
    )
    return call


def _build_score_call():
    info = plsc.get_sparse_core_info()
    nc, ns = info.num_cores, info.num_subcores
    nw = nc * ns
    bpw = _B // nw
    nchunk = bpw // _CHUNK

    mesh = plsc.VectorSubcoreMesh(core_axis_name="c", subcore_axis_name="s")

    def body(user_hbm, pos_hbm, neg_hbm, utab_hbm, itab_hbm,
             pos_out, neg_out,
             uidx_v, pidx_v, nidx_v, ubidx_v, pbidx_v, nbidx_v,
             ubuf_v, pbuf_v, nbuf_v,
             posbuf_v, negbuf_v, sem):
        wid = lax.axis_index("s") * nc + lax.axis_index("c")
        ibase = wid * nchunk

        pltpu.sync_copy(user_hbm.at[pl.ds(ibase, nchunk)], uidx_v)
        pltpu.sync_copy(pos_hbm.at[pl.ds(ibase, nchunk)], pidx_v)
        pltpu.sync_copy(neg_hbm.at[pl.ds(ibase, nchunk)], nidx_v)

        def chunk(j, carry):
            for d in range(_D):
                for k in range(_CHUNK // 16):
                    sl = pl.ds(k * 16, 16)
                    ubidx_v[d, sl] = uidx_v[j, sl] + d * _LANES
                    pbidx_v[d, sl] = pidx_v[j, sl] + d * _LANES
                    nbidx_v[d, sl] = nidx_v[j, sl] + d * _LANES
            copies = []
            for d in range(_D):
                copies.append(pltpu.async_copy(
                    utab_hbm.at[ubidx_v.at[d]], ubuf_v.at[d], sem))
                copies.append(pltpu.async_copy(
                    itab_hbm.at[pbidx_v.at[d]], pbuf_v.at[d], sem))
                copies.append(pltpu.async_copy(
                    itab_hbm.at[nbidx_v.at[d]], nbuf_v.at[d], sem))
            for c in copies:
                c.wait()

            for k in range(_CHUNK // 16):
                sl = pl.ds(k * 16, 16)
                accp = jnp.zeros((16,), jnp.float32)
                accn = jnp.zeros((16,), jnp.float32)
                for d in range(_D):
                    u = ubuf_v[d, sl]
                    accp = accp + u * pbuf_v[d, sl]
                    accn = accn + u * nbuf_v[d, sl]
                posbuf_v[pl.ds(j * _CHUNK + k * 16, 16)] = accp
                negbuf_v[pl.ds(j * _CHUNK + k * 16, 16)] = accn
            return carry

        lax.fori_loop(0, nchunk, chunk, 0)

        obase = wid * bpw
        pltpu.sync_copy(posbuf_v, pos_out.at[pl.ds(obase, bpw)])
        pltpu.sync_copy(negbuf_v, neg_out.at[pl.ds(obase, bpw)])

    call = pl.kernel(
        body,
        out_type=(jax.ShapeDtypeStruct((_B,), jnp.float32),
                  jax.ShapeDtypeStruct((_B,), jnp.float32)),
        mesh=mesh,
        scratch_types=[
            pltpu.VMEM((_B // _CHUNK // 32, _CHUNK), jnp.int32),
            pltpu.VMEM((_B // _CHUNK // 32, _CHUNK), jnp.int32),
            pltpu.VMEM((_B // _CHUNK // 32, _CHUNK), jnp.int32),
            pltpu.VMEM((_D, _CHUNK), jnp.int32),
            pltpu.VMEM((_D, _CHUNK), jnp.int32),
            pltpu.VMEM((_D, _CHUNK), jnp.int32),
            pltpu.VMEM((_D, _CHUNK), jnp.float32),
            pltpu.VMEM((_D, _CHUNK), jnp.float32),
            pltpu.VMEM((_D, _CHUNK), jnp.float32),
            pltpu.VMEM((_B // 32,), jnp.float32),
            pltpu.VMEM((_B // 32,), jnp.float32),
            pltpu.SemaphoreType.DMA,
        ],
        compiler_params=pltpu.CompilerParams(
            needs_layout_passes=False, use_tc_tiling_on_sc=False),
    )
    return call


def kernel(user, pos_item, neg_item, user_table, item_table):
    detile = _build_detile_call()
    score = _build_score_call()
    ut = jnp.swapaxes(user_table, 0, 1)
    it = jnp.swapaxes(item_table, 0, 1)
    ulin, ilin = detile(ut, it)
    u2 = user.astype(jnp.int32).reshape(_B // _CHUNK, _CHUNK)
    p2 = pos_item.astype(jnp.int32).reshape(_B // _CHUNK, _CHUNK)
    n2 = neg_item.astype(jnp.int32).reshape(_B // _CHUNK, _CHUNK)
    return score(u2, p2, n2, ulin, ilin)


# speedup vs baseline: 15.9176x; 1.0277x over previous
import jax
import jax.numpy as jnp
from jax import lax
from jax.experimental import pallas as pl
from jax.experimental.pallas import tpu as pltpu, tpu_sc as plsc

_B = 16384
_D = 32
_CHUNK = 128
_NROW = 1000001
_NCOL = 7813
_LANES = _NCOL * 128
_SUPER = 12


def _build_detile_call():
    info = plsc.get_sparse_core_info()
    nc, ns = info.num_cores, info.num_subcores
    nw = nc * ns
    mesh = plsc.VectorSubcoreMesh(core_axis_name="c", subcore_axis_name="s")
    sw = _SUPER * 128

    def body(utab_hbm, itab_hbm, ulin_out, ilin_out, blk_a, blk_b, tblk,
             rsem, wsem_a, wsem_b):
        wid = lax.axis_index("s") * nc + lax.axis_index("c")
        lo = wid * _NCOL // nw
        hi = (wid + 1) * _NCOL // nw
        nsup = (hi - lo) // _SUPER

        dummy_big = utab_hbm.at[:, pl.ds(0, sw)]
        dummy_small = utab_hbm.at[:, pl.ds(0, 128)]
        max_base = (_NCOL - _SUPER) * 128
        npair = nsup // 2

        for tab, out in ((utab_hbm, ulin_out), (itab_hbm, ilin_out)):
            pltpu.async_copy(tab.at[:, pl.ds(lo * 128, sw)], blk_a, rsem)

            def pair_step(i, carry):
                base0 = (lo + 2 * i * _SUPER) * 128
                base1 = base0 + sw
                base2 = jnp.minimum(base0 + 2 * sw, max_base)
                pltpu.make_async_copy(dummy_big, blk_a, rsem).wait()
                for d in range(_D):
                    pltpu.async_copy(
                        blk_a.at[d], out.at[pl.ds(d * _LANES + base0, sw)],
                        wsem_a)
                @pl.when(i > 0)
                def _():
                    pltpu.make_async_copy(dummy_big, blk_b, wsem_b).wait()
                pltpu.async_copy(tab.at[:, pl.ds(base1, sw)], blk_b, rsem)
                pltpu.make_async_copy(dummy_big, blk_b, rsem).wait()
                for d in range(_D):
                    pltpu.async_copy(
                        blk_b.at[d], out.at[pl.ds(d * _LANES + base1, sw)],
                        wsem_b)
                pltpu.make_async_copy(dummy_big, blk_a, wsem_a).wait()
                pltpu.async_copy(tab.at[:, pl.ds(base2, sw)], blk_a, rsem)
                return carry

            lax.fori_loop(0, npair, pair_step, 0)
            pltpu.make_async_copy(dummy_big, blk_a, rsem).wait()
            pltpu.make_async_copy(dummy_big, blk_b, wsem_b).wait()

            def tail_step(i, carry):
                c = lo + npair * 2 * _SUPER + i
                base = c * 128
                pltpu.async_copy(
                    tab.at[:, pl.ds(base, 128)], tblk, rsem).wait()
                for d in range(_D):
                    pltpu.async_copy(
                        tblk.at[d], out.at[pl.ds(d * _LANES + base, 128)],
                        wsem_a)
                pltpu.make_async_copy(dummy_small, tblk, wsem_a).wait()
                return carry

            lax.fori_loop(0, (hi - lo) - npair * 2 * _SUPER, tail_step, 0)

    call = pl.kernel(
        body,
        out_type=(jax.ShapeDtypeStruct((_D * _LANES,), jnp.float32),
                  jax.ShapeDtypeStruct((_D * _LANES,), jnp.float32)),
        mesh=mesh,
        scratch_types=[
            pltpu.VMEM((_D, sw), jnp.float32),
            pltpu.VMEM((_D, sw), jnp.float32),
            pltpu.VMEM((_D, 128), jnp.float32),
            pltpu.SemaphoreType.DMA,
            pltpu.SemaphoreType.DMA,
            pltpu.SemaphoreType.DMA,
        ],
        compiler_params=pltpu.CompilerParams(
            needs_layout_passes=False, use_tc_tiling_on_sc=True,
            disable_bounds_checks=True),
    )
    return call


def _build_score_call():
    info = plsc.get_sparse_core_info()
    nc, ns = info.num_cores, info.num_subcores
    nw = nc * ns
    bpw = _B // nw
    nchunk = bpw // _CHUNK

    mesh = plsc.VectorSubcoreMesh(core_axis_name="c", subcore_axis_name="s")

    def body(user_hbm, pos_hbm, neg_hbm, utab_hbm, itab_hbm,
             pos_out, neg_out,
             uidx_v, pidx_v, nidx_v, ubidx_v, pbidx_v, nbidx_v,
             ubuf_v, pbuf_v, nbuf_v,
             posbuf_v, negbuf_v, sem):
        wid = lax.axis_index("s") * nc + lax.axis_index("c")
        ibase = wid * nchunk

        pltpu.sync_copy(user_hbm.at[pl.ds(ibase, nchunk)], uidx_v)
        pltpu.sync_copy(pos_hbm.at[pl.ds(ibase, nchunk)], pidx_v)
        pltpu.sync_copy(neg_hbm.at[pl.ds(ibase, nchunk)], nidx_v)

        def chunk(j, carry):
            for d in range(_D):
                for k in range(_CHUNK // 16):
                    sl = pl.ds(k * 16, 16)
                    ubidx_v[d, sl] = uidx_v[j, sl] + d * _LANES
                    pbidx_v[d, sl] = pidx_v[j, sl] + d * _LANES
                    nbidx_v[d, sl] = nidx_v[j, sl] + d * _LANES
            copies = []
            for d in range(_D):
                copies.append(pltpu.async_copy(
                    utab_hbm.at[ubidx_v.at[d]], ubuf_v.at[d], sem))
                copies.append(pltpu.async_copy(
                    itab_hbm.at[pbidx_v.at[d]], pbuf_v.at[d], sem))
                copies.append(pltpu.async_copy(
                    itab_hbm.at[nbidx_v.at[d]], nbuf_v.at[d], sem))
            for c in copies:
                c.wait()

            for k in range(_CHUNK // 16):
                sl = pl.ds(k * 16, 16)
                accp = jnp.zeros((16,), jnp.float32)
                accn = jnp.zeros((16,), jnp.float32)
                for d in range(_D):
                    u = ubuf_v[d, sl]
                    accp = accp + u * pbuf_v[d, sl]
                    accn = accn + u * nbuf_v[d, sl]
                posbuf_v[pl.ds(j * _CHUNK + k * 16, 16)] = accp
                negbuf_v[pl.ds(j * _CHUNK + k * 16, 16)] = accn
            return carry

        lax.fori_loop(0, nchunk, chunk, 0)

        obase = wid * bpw
        pltpu.sync_copy(posbuf_v, pos_out.at[pl.ds(obase, bpw)])
        pltpu.sync_copy(negbuf_v, neg_out.at[pl.ds(obase, bpw)])

    call = pl.kernel(
        body,
        out_type=(jax.ShapeDtypeStruct((_B,), jnp.float32),
                  jax.ShapeDtypeStruct((_B,), jnp.float32)),
        mesh=mesh,
        scratch_types=[
            pltpu.VMEM((_B // _CHUNK // 32, _CHUNK), jnp.int32),
            pltpu.VMEM((_B // _CHUNK // 32, _CHUNK), jnp.int32),
            pltpu.VMEM((_B // _CHUNK // 32, _CHUNK), jnp.int32),
            pltpu.VMEM((_D, _CHUNK), jnp.int32),
            pltpu.VMEM((_D, _CHUNK), jnp.int32),
            pltpu.VMEM((_D, _CHUNK), jnp.int32),
            pltpu.VMEM((_D, _CHUNK), jnp.float32),
            pltpu.VMEM((_D, _CHUNK), jnp.float32),
            pltpu.VMEM((_D, _CHUNK), jnp.float32),
            pltpu.VMEM((_B // 32,), jnp.float32),
            pltpu.VMEM((_B // 32,), jnp.float32),
            pltpu.SemaphoreType.DMA,
        ],
        compiler_params=pltpu.CompilerParams(
            needs_layout_passes=False, use_tc_tiling_on_sc=False),
    )
    return call


def kernel(user, pos_item, neg_item, user_table, item_table):
    detile = _build_detile_call()
    score = _build_score_call()
    ut = jnp.swapaxes(user_table, 0, 1)
    it = jnp.swapaxes(item_table, 0, 1)
    ulin, ilin = detile(ut, it)
    u2 = user.astype(jnp.int32).reshape(_B // _CHUNK, _CHUNK)
    p2 = pos_item.astype(jnp.int32).reshape(_B // _CHUNK, _CHUNK)
    n2 = neg_item.astype(jnp.int32).reshape(_B // _CHUNK, _CHUNK)
    return score(u2, p2, n2, ulin, ilin)
